# baseline (speedup 1.0000x reference)
import jax
import jax.numpy as jnp
from jax import lax
from jax.experimental import pallas as pl
from jax.experimental.pallas import tpu as pltpu
from jax.experimental.pallas import tpu_sc as plsc

_DEPTH = 1000
_ROWS = 4096 * 26
_NSC = 2
_RPC = _ROWS // _NSC
_WIN = 512
_NWIN = _RPC // _WIN
_WROWS = _WIN * _DEPTH // 128


def _make_kernel():
    smesh = plsc.ScalarSubcoreMesh(axis_name="c", num_cores=_NSC)

    @pl.kernel(
        out_type=jax.ShapeDtypeStruct((_ROWS * _DEPTH // 128, 128), jnp.float32),
        mesh=smesh,
        scratch_types=[
            pltpu.VMEM_SHARED((_WROWS, 128), jnp.float32),
        ],
        compiler_params=pltpu.CompilerParams(needs_layout_passes=False),
    )
    def zerofill(zsrc_hbm, out_hbm, zshared):
        cid = lax.axis_index("c")
        base = cid * _RPC * _DEPTH // 128
        pltpu.sync_copy(zsrc_hbm, zshared)

        def run(zsem):
            def zfire(m, carry):
                off = pl.multiple_of(base + m * _WROWS, 8)
                dst = out_hbm.at[pl.ds(off, _WROWS)]
                pltpu.async_copy(zshared, dst, zsem)
                return carry

            lax.fori_loop(0, _NWIN, zfire, 0)

            def zdrain(m, carry):
                off = pl.multiple_of(base + m * _WROWS, 8)
                dst = out_hbm.at[pl.ds(off, _WROWS)]
                pltpu.make_async_copy(zshared, dst, zsem).wait()
                return carry

            lax.fori_loop(0, _NWIN, zdrain, 0)

        pl.run_scoped(run, pltpu.SemaphoreType.DMA)

    return zerofill


_onehot = _make_kernel()


def kernel(inputs):
    zsrc = jnp.zeros((_WROWS, 128), jnp.float32)
    flat = _onehot(zsrc)
    return flat.reshape(inputs.shape[0], inputs.shape[1], _DEPTH)


# speedup vs baseline: 1.1842x; 1.1842x over previous
import functools

import jax
import jax.numpy as jnp
from jax import lax
from jax.experimental import pallas as pl
from jax.experimental.pallas import tpu as pltpu
from jax.experimental.pallas import tpu_sc as plsc

_DEPTH = 1000
_ROWS = 4096 * 26
_NW = 32
_RPW = _ROWS // _NW
_R = 16
_CH = _RPW // _R
_NBUF = 4


def _make_kernel():
    mesh = plsc.VectorSubcoreMesh(core_axis_name="c", subcore_axis_name="s")

    @functools.partial(
        pl.kernel,
        mesh=mesh,
        out_type=jax.ShapeDtypeStruct((_ROWS * _DEPTH,), jnp.float32),
        scratch_types=[
            pltpu.VMEM((_RPW,), jnp.int32),
        ]
        + [pltpu.VMEM((_R * _DEPTH,), jnp.float32) for _ in range(_NBUF)]
        + [pltpu.SemaphoreType.DMA for _ in range(_NBUF)],
        compiler_params=pltpu.CompilerParams(needs_layout_passes=False),
    )
    def onehot(idx_hbm, out_hbm, idx_v, *bufs_sems):
        bufs = bufs_sems[:_NBUF]
        sems = bufs_sems[_NBUF:]
        wid = lax.axis_index("s") * 2 + lax.axis_index("c")
        base_row = wid * _RPW
        pltpu.sync_copy(idx_hbm.at[pl.ds(base_row, _RPW)], idx_v)

        zeros = jnp.zeros((16,), jnp.float32)
        ones = jnp.ones((16,), jnp.float32)
        lane = lax.iota(jnp.int32, 16)

        def zbody(j, carry):
            for b in range(_NBUF):
                bufs[b][pl.ds(j * 16, 16)] = zeros
            return carry

        lax.fori_loop(0, _R * _DEPTH // 16, zbody, 0)

        def scatter(c, buf, val):
            idxv = idx_v[pl.ds(c * _R, 16)]
            pos = lane * _DEPTH + idxv
            plsc.store_scatter(buf, [pos], val)

        def out_slice(c):
            return out_hbm.at[pl.ds((base_row + c * _R) * _DEPTH, _R * _DEPTH)]

        for b in range(_NBUF):
            scatter(b, bufs[b], ones)
            pltpu.async_copy(bufs[b], out_slice(b), sems[b])

        def obody(t, carry):
            for b in range(_NBUF):
                c = (t + 1) * _NBUF + b
                pltpu.make_async_copy(bufs[b], out_slice(c - _NBUF), sems[b]).wait()
                scatter(c - _NBUF, bufs[b], zeros)
                scatter(c, bufs[b], ones)
                pltpu.async_copy(bufs[b], out_slice(c), sems[b])
            return carry

        lax.fori_loop(0, _CH // _NBUF - 1, obody, 0)

        for b in range(_NBUF):
            c_last = _CH - _NBUF + b
            pltpu.make_async_copy(bufs[b], out_slice(c_last), sems[b]).wait()

    return onehot


_onehot = _make_kernel()


def kernel(inputs):
    idx = jnp.asarray(inputs, jnp.int32).reshape(-1)
    flat = _onehot(idx)
    return flat.reshape(inputs.shape[0], inputs.shape[1], _DEPTH)
